# baseline (speedup 1.0000x reference)
import functools
import jax
import jax.numpy as jnp
from jax import lax
from jax.experimental import pallas as pl
from jax.experimental.pallas import tpu as pltpu
from jax.experimental.pallas import tpu_sc as plsc

DIM = 64


def kernel(matrix, W_struct, W_value):
    B, L, K = matrix.shape
    LK = L * K
    TB = B // 128
    n_chunks_total = LK * TB
    info = plsc.get_sparse_core_info()
    NC, NS = info.num_cores, info.num_subcores
    NW = NC * NS
    NCH = n_chunks_total // NW
    ROWS_LK = DIM * TB

    idx16 = matrix.reshape(B * LK // 16, 16)

    mesh = plsc.VectorSubcoreMesh(core_axis_name="c", subcore_axis_name="s")

    @functools.partial(
        pl.kernel,
        mesh=mesh,
        compiler_params=pltpu.CompilerParams(
            needs_layout_passes=False, use_tc_tiling_on_sc=False),
        out_type=jax.ShapeDtypeStruct((LK * ROWS_LK, 128), jnp.float32),
        scratch_types=[
            pltpu.VMEM((2, 128), jnp.int32),
            pltpu.VMEM((2, 128, 16), jnp.int32),
            pltpu.VMEM((2, 128), jnp.int32),
            pltpu.VMEM((2, DIM), jnp.int32),
            pltpu.VMEM((2, 128, DIM), jnp.float32),
            pltpu.VMEM((2, DIM, 128), jnp.float32),
            pltpu.SemaphoreType.DMA,
            pltpu.SemaphoreType.DMA,
            pltpu.SemaphoreType.DMA,
            pltpu.SemaphoreType.DMA,
            pltpu.SemaphoreType.DMA,
            pltpu.SemaphoreType.DMA,
        ],
    )
    def kern(idx_hbm, ws_hbm, wv_hbm, out_hbm,
             gref, land, dense, rlist, buf, bufT,
             gi0, gi1, ts0, ts1, ss0, ss1):
        wid = lax.axis_index("s") * NC + lax.axis_index("c")
        q0 = wid * NCH
        iota = lax.iota(jnp.int32, 16)
        gi = (gi0, gi1)
        ts = (ts0, ts1)
        ss = (ss0, ss1)

        tb_bits = TB.bit_length() - 1

        def chunk_coords(v):
            q = q0 + v
            lk = lax.shift_right_logical(q, tb_bits)
            tb = lax.bitwise_and(q, TB - 1)
            return lk, tb

        def start_idx_gather(v, b):
            lk, tb = chunk_coords(v)
            base = tb * 128 * LK + lk
            for j in range(8):
                p = base + LK * (iota + 16 * j)
                gref[b, pl.ds(16 * j, 16)] = lax.shift_right_logical(p, 4)
            pltpu.async_copy(idx_hbm.at[gref.at[b]], land.at[b], gi[b])

        def wait_idx_gather(b):
            pltpu.make_async_copy(idx_hbm.at[gref.at[b]], land.at[b], gi[b]).wait()

        def extract_and_table_gather(v, b):
            lk, tb = chunk_coords(v)
            base = tb * 128 * LK + lk
            for j in range(8):
                cc = iota + 16 * j
                p = base + LK * cc
                off = lax.bitwise_and(p, 15)
                val = plsc.load_gather(land.at[b], [cc, off])
                dense[b, pl.ds(16 * j, 16)] = val
            k_col = lk % K
            parity = lax.bitwise_and(k_col, 1)

            @pl.when(parity == 0)
            def _():
                pltpu.async_copy(ws_hbm.at[dense.at[b]], buf.at[b], ts[b])

            @pl.when(parity == 1)
            def _():
                pltpu.async_copy(wv_hbm.at[dense.at[b]], buf.at[b], ts[b])

        def wait_table_gather(b):
            pltpu.make_async_copy(ws_hbm.at[dense.at[b]], buf.at[b], ts[b]).wait()

        def transpose_and_scatter(v, b):
            lk, tb = chunk_coords(v)

            @pl.loop(0, DIM)
            def _(d):
                dvec = lax.broadcast(d, (16,))
                for j in range(8):
                    vals = plsc.load_gather(buf.at[b], [iota + 16 * j, dvec])
                    bufT[b, d, pl.ds(16 * j, 16)] = vals

            rbase = lk * ROWS_LK + tb * 8
            for j in range(DIM // 16):
                d = iota + 16 * j
                rr = rbase + 64 * lax.shift_right_logical(d, 3) + lax.bitwise_and(d, 7)
                rlist[b, pl.ds(16 * j, 16)] = rr
            pltpu.async_copy(bufT.at[b], out_hbm.at[rlist.at[b]], ss[b])

        def wait_scatter(b):
            pltpu.make_async_copy(bufT.at[b], out_hbm.at[rlist.at[b]], ss[b]).wait()

        def visit(v, b, with_prev, with_scatter_wait, prefetch=True):
            wait_idx_gather(b)
            extract_and_table_gather(v, b)
            if prefetch:
                start_idx_gather(v + 2, b)
            if with_prev:
                wait_table_gather(1 - b)
                if with_scatter_wait:
                    wait_scatter(1 - b)
                transpose_and_scatter(v - 1, 1 - b)

        start_idx_gather(0, 0)
        start_idx_gather(1, 1)
        visit(0, 0, with_prev=False, with_scatter_wait=False)
        visit(1, 1, with_prev=True, with_scatter_wait=False)
        visit(2, 0, with_prev=True, with_scatter_wait=False)
        visit(3, 1, with_prev=True, with_scatter_wait=True)

        @pl.loop(4, NCH - 2, step=2)
        def _(v0):
            for bb in range(2):
                visit(v0 + bb, bb, with_prev=True, with_scatter_wait=True)

        visit(NCH - 2, 0, with_prev=True, with_scatter_wait=True, prefetch=False)
        visit(NCH - 1, 1, with_prev=True, with_scatter_wait=True, prefetch=False)
        wait_table_gather(1)
        wait_scatter(1)
        transpose_and_scatter(NCH - 1, 1)
        wait_scatter(0)
        wait_scatter(1)

    out = kern(idx16, W_struct, W_value)
    out = out.reshape(L, K, DIM // 8, TB, 8, 128)
    out = out.transpose(3, 5, 0, 1, 2, 4)
    return out.reshape(B, L, K, DIM)


# speedup vs baseline: 2.4935x; 2.4935x over previous
import functools
import jax
import jax.numpy as jnp
from jax import lax
from jax.experimental import pallas as pl
from jax.experimental.pallas import tpu as pltpu
from jax.experimental.pallas import tpu_sc as plsc

DIM = 64


def kernel(matrix, W_struct, W_value):
    B, L, K = matrix.shape
    LK = L * K
    TB = B // 128
    n_chunks_total = LK * TB
    info = plsc.get_sparse_core_info()
    NC, NS = info.num_cores, info.num_subcores
    NW = NC * NS
    NCH = n_chunks_total // NW
    ROWS_LK = DIM * TB

    idx16 = matrix.reshape(B * LK // 16, 16)

    mesh = plsc.VectorSubcoreMesh(core_axis_name="c", subcore_axis_name="s")

    @functools.partial(
        pl.kernel,
        mesh=mesh,
        compiler_params=pltpu.CompilerParams(
            needs_layout_passes=False, use_tc_tiling_on_sc=False),
        out_type=jax.ShapeDtypeStruct((LK * ROWS_LK, 128), jnp.float32),
        scratch_types=[
            pltpu.VMEM((2, 128), jnp.int32),
            pltpu.VMEM((2, 128, 16), jnp.int32),
            pltpu.VMEM((2, 128), jnp.int32),
            pltpu.VMEM((2, DIM), jnp.int32),
            pltpu.VMEM((2, 128, DIM), jnp.float32),
            pltpu.VMEM((2, DIM, 128), jnp.float32),
            pltpu.SemaphoreType.DMA,
            pltpu.SemaphoreType.DMA,
            pltpu.SemaphoreType.DMA,
            pltpu.SemaphoreType.DMA,
            pltpu.SemaphoreType.DMA,
            pltpu.SemaphoreType.DMA,
        ],
    )
    def kern(idx_hbm, ws_hbm, wv_hbm, out_hbm,
             gref, land, dense, rlist, buf, bufT,
             gi0, gi1, ts0, ts1, ss0, ss1):
        wid = lax.axis_index("s") * NC + lax.axis_index("c")
        q0 = wid * NCH
        iota = lax.iota(jnp.int32, 16)
        gi = (gi0, gi1)
        ts = (ts0, ts1)
        ss = (ss0, ss1)

        tb_bits = TB.bit_length() - 1

        def chunk_coords(v):
            q = q0 + v
            lk = lax.shift_right_logical(q, tb_bits)
            tb = lax.bitwise_and(q, TB - 1)
            return lk, tb

        def start_idx_gather(v, b):
            lk, tb = chunk_coords(v)
            base = tb * 128 * LK + lk
            for j in range(8):
                p = base + LK * (iota + 16 * j)
                gref[b, pl.ds(16 * j, 16)] = lax.shift_right_logical(p, 4)
            pltpu.async_copy(idx_hbm.at[gref.at[b]], land.at[b], gi[b])

        def wait_idx_gather(b):
            pltpu.make_async_copy(idx_hbm.at[gref.at[b]], land.at[b], gi[b]).wait()

        def extract_and_table_gather(v, b):
            lk, tb = chunk_coords(v)
            base = tb * 128 * LK + lk
            for j in range(8):
                cc = iota + 16 * j
                p = base + LK * cc
                off = lax.bitwise_and(p, 15)
                val = plsc.load_gather(land.at[b], [cc, off])
                dense[b, pl.ds(16 * j, 16)] = val
            k_col = lk % K
            parity = lax.bitwise_and(k_col, 1)

            @pl.when(parity == 0)
            def _():
                pltpu.async_copy(ws_hbm.at[dense.at[b]], buf.at[b], ts[b])

            @pl.when(parity == 1)
            def _():
                pltpu.async_copy(wv_hbm.at[dense.at[b]], buf.at[b], ts[b])

        def wait_table_gather(b):
            pltpu.make_async_copy(ws_hbm.at[dense.at[b]], buf.at[b], ts[b]).wait()

        def transpose_and_scatter(v, b):
            lk, tb = chunk_coords(v)

            @pl.loop(0, 16)
            def _(t):
                perm = lax.bitwise_and(iota + t, 15)
                for jc in range(8):
                    cvec = iota + 16 * jc
                    for jd in range(DIM // 16):
                        dcol = 16 * jd + perm
                        vals = plsc.load_gather(buf.at[b], [cvec, dcol])
                        plsc.store_scatter(bufT.at[b], [dcol, cvec], vals)

            rbase = lk * ROWS_LK + tb * 8
            for j in range(DIM // 16):
                d = iota + 16 * j
                rr = rbase + 64 * lax.shift_right_logical(d, 3) + lax.bitwise_and(d, 7)
                rlist[b, pl.ds(16 * j, 16)] = rr
            pltpu.async_copy(bufT.at[b], out_hbm.at[rlist.at[b]], ss[b])

        def wait_scatter(b):
            pltpu.make_async_copy(bufT.at[b], out_hbm.at[rlist.at[b]], ss[b]).wait()

        def visit(v, b, with_prev, with_scatter_wait, prefetch=True):
            wait_idx_gather(b)
            extract_and_table_gather(v, b)
            if prefetch:
                start_idx_gather(v + 2, b)
            if with_prev:
                wait_table_gather(1 - b)
                if with_scatter_wait:
                    wait_scatter(1 - b)
                transpose_and_scatter(v - 1, 1 - b)

        start_idx_gather(0, 0)
        start_idx_gather(1, 1)
        visit(0, 0, with_prev=False, with_scatter_wait=False)
        visit(1, 1, with_prev=True, with_scatter_wait=False)
        visit(2, 0, with_prev=True, with_scatter_wait=False)
        visit(3, 1, with_prev=True, with_scatter_wait=True)

        @pl.loop(4, NCH - 2, step=2)
        def _(v0):
            for bb in range(2):
                visit(v0 + bb, bb, with_prev=True, with_scatter_wait=True)

        visit(NCH - 2, 0, with_prev=True, with_scatter_wait=True, prefetch=False)
        visit(NCH - 1, 1, with_prev=True, with_scatter_wait=True, prefetch=False)
        wait_table_gather(1)
        wait_scatter(1)
        transpose_and_scatter(NCH - 1, 1)
        wait_scatter(0)
        wait_scatter(1)

    out = kern(idx16, W_struct, W_value)
    out = out.reshape(L, K, DIM // 8, TB, 8, 128)
    out = out.transpose(3, 5, 0, 1, 2, 4)
    return out.reshape(B, L, K, DIM)
